# baseline (speedup 1.0000x reference)
import functools

import jax
import jax.numpy as jnp
from jax import lax
from jax.experimental import pallas as pl
from jax.experimental.pallas import tpu as pltpu
from jax.experimental.pallas import tpu_sc as plsc

VOCAB = 1000000
EMBED_DIM = 64
PROJ_DIM = 32
FEAT = 10
OUT_DIM = 96
B = 4096
L = 200
BL = B * L

NUM_CORES = 2
NUM_SUBCORES = 16
NW = NUM_CORES * NUM_SUBCORES
ITEMS_PER_W = BL // NW
CB = 512
NCHUNK = ITEMS_PER_W // CB

_mesh = plsc.VectorSubcoreMesh(core_axis_name="c", subcore_axis_name="s")


@functools.partial(
    pl.kernel,
    mesh=_mesh,
    compiler_params=pltpu.CompilerParams(
        needs_layout_passes=False, use_tc_tiling_on_sc=False),
    out_type=jax.ShapeDtypeStruct((BL * OUT_DIM,), jnp.float32),
    scratch_types=[
        pltpu.VMEM((CB * FEAT,), jnp.float32),
        pltpu.VMEM((CB,), jnp.int32),
        pltpu.VMEM((CB, EMBED_DIM), jnp.float32),
        pltpu.VMEM((CB * OUT_DIM,), jnp.float32),
        pltpu.VMEM(((FEAT - 1) * PROJ_DIM,), jnp.float32),
        pltpu.SemaphoreType.DMA,
    ],
)
def _encode(x_hbm, tab_hbm, ws_hbm, out_hbm, xv, idv, rows, buf, wv, sem):
    wid = lax.axis_index("s") * NUM_CORES + lax.axis_index("c")
    pltpu.sync_copy(ws_hbm, wv)
    lane10 = lax.iota(jnp.int32, 16) * FEAT

    wregs = [wv[pl.ds(k * PROJ_DIM + h * 16, 16)]
             for k in range(FEAT - 1) for h in range(2)]

    def chunk_body(c, _):
        base = (wid * ITEMS_PER_W + c * CB)
        pltpu.sync_copy(x_hbm.at[pl.ds(base * FEAT, CB * FEAT)], xv)

        def id_body(j, _):
            idxs = j * (16 * FEAT) + lane10
            idv[pl.ds(j * 16, 16)] = plsc.load_gather(xv, [idxs]).astype(jnp.int32)
            return 0
        lax.fori_loop(0, CB // 16, id_body, 0)

        pltpu.async_copy(tab_hbm.at[idv], rows, sem).wait()

        def item_body(i, _):
            obase = i * OUT_DIM
            for t in range(EMBED_DIM // 16):
                buf[pl.ds(obase + t * 16, 16)] = rows[i, pl.ds(t * 16, 16)]
            accs = [None, None]
            for k in range(FEAT - 1):
                s = plsc.load_gather(
                    xv, [jnp.full((16,), i * FEAT + 1 + k, jnp.int32)])
                for h in range(2):
                    t = s * wregs[2 * k + h]
                    accs[h] = t if accs[h] is None else accs[h] + t
            for h in range(2):
                buf[pl.ds(obase + EMBED_DIM + h * 16, 16)] = accs[h]
            return 0
        lax.fori_loop(0, CB, item_body, 0)

        pltpu.sync_copy(buf, out_hbm.at[pl.ds(base * OUT_DIM, CB * OUT_DIM)])
        return 0

    lax.fori_loop(0, NCHUNK, chunk_body, 0)


def kernel(x, table, W):
    x_flat = x.reshape(BL * FEAT)
    ws = (W.T / 255.0).reshape((FEAT - 1) * PROJ_DIM).astype(jnp.float32)
    out = _encode(x_flat, table, ws)
    return out.reshape(B, L, OUT_DIM)


# speedup vs baseline: 1.1936x; 1.1936x over previous
import functools

import jax
import jax.numpy as jnp
from jax import lax
from jax.experimental import pallas as pl
from jax.experimental.pallas import tpu as pltpu
from jax.experimental.pallas import tpu_sc as plsc

VOCAB = 1000000
EMBED_DIM = 64
PROJ_DIM = 32
FEAT = 10
OUT_DIM = 96
B = 4096
L = 200
BL = B * L

NUM_CORES = 2
NUM_SUBCORES = 16
NW = NUM_CORES * NUM_SUBCORES
ITEMS_PER_W = BL // NW
CB = 512
NCHUNK = ITEMS_PER_W // CB

_mesh = plsc.VectorSubcoreMesh(core_axis_name="c", subcore_axis_name="s")


@functools.partial(
    pl.kernel,
    mesh=_mesh,
    compiler_params=pltpu.CompilerParams(
        needs_layout_passes=False, use_tc_tiling_on_sc=False),
    out_type=jax.ShapeDtypeStruct((BL, OUT_DIM), jnp.float32),
    scratch_types=[
        pltpu.VMEM((2, CB * FEAT), jnp.float32),
        pltpu.VMEM((2, CB), jnp.int32),
        pltpu.VMEM((2, CB, EMBED_DIM), jnp.float32),
        pltpu.VMEM((2, CB, PROJ_DIM), jnp.float32),
        pltpu.VMEM(((FEAT - 1) * PROJ_DIM,), jnp.float32),
        pltpu.SemaphoreType.DMA((2,)),
        pltpu.SemaphoreType.DMA((2,)),
        pltpu.SemaphoreType.DMA((2,)),
    ],
)
def _encode(x_hbm, tab_hbm, ws_hbm, out_hbm,
            xv, idv, rows, projv, wv, gsem, rsem, psem):
    wid = lax.axis_index("s") * NUM_CORES + lax.axis_index("c")
    pltpu.sync_copy(ws_hbm, wv)
    lane10 = lax.iota(jnp.int32, 16) * FEAT

    wregs = [wv[pl.ds(k * PROJ_DIM + h * 16, 16)]
             for k in range(FEAT - 1) for h in range(2)]

    def chunk_base(c):
        return wid * ITEMS_PER_W + c * CB

    def stage1(c, k):
        base = chunk_base(c)
        pltpu.sync_copy(x_hbm.at[pl.ds(base * FEAT, CB * FEAT)], xv.at[k])

        def id_body(j, _):
            idxs = j * (16 * FEAT) + lane10
            idv[k, pl.ds(j * 16, 16)] = (
                plsc.load_gather(xv.at[k], [idxs]).astype(jnp.int32))
            return 0
        lax.fori_loop(0, CB // 16, id_body, 0)
        pltpu.make_async_copy(
            tab_hbm.at[idv.at[k]], rows.at[k], gsem.at[k]).start()

    def wait_gather(k):
        pltpu.make_async_copy(
            tab_hbm.at[idv.at[k]], rows.at[k], gsem.at[k]).wait()

    def wait_outs(k):
        pltpu.make_async_copy(
            rows.at[k],
            out_hbm.at[pl.ds(0, CB), pl.ds(0, EMBED_DIM)],
            rsem.at[k]).wait()
        pltpu.make_async_copy(
            projv.at[k],
            out_hbm.at[pl.ds(0, CB), pl.ds(EMBED_DIM, PROJ_DIM)],
            psem.at[k]).wait()

    def compute(c, k):
        def item_body(i, _):
            accs = [None, None]
            for f in range(FEAT - 1):
                s = plsc.load_gather(
                    xv.at[k], [jnp.full((16,), i * FEAT + 1 + f, jnp.int32)])
                for h in range(2):
                    t = s * wregs[2 * f + h]
                    accs[h] = t if accs[h] is None else accs[h] + t
            for h in range(2):
                projv[k, i, pl.ds(h * 16, 16)] = accs[h]
            return 0
        lax.fori_loop(0, CB, item_body, 0, unroll=8)

    def fire_outs(c, k):
        base = chunk_base(c)
        pltpu.make_async_copy(
            rows.at[k],
            out_hbm.at[pl.ds(base, CB), pl.ds(0, EMBED_DIM)],
            rsem.at[k]).start()
        pltpu.make_async_copy(
            projv.at[k],
            out_hbm.at[pl.ds(base, CB), pl.ds(EMBED_DIM, PROJ_DIM)],
            psem.at[k]).start()

    stage1(0, 0)

    stage1(1, 1)
    wait_gather(0)
    compute(0, 0)
    fire_outs(0, 0)

    def pair_body(j, _):
        for p in range(2):
            c = 2 * j + 1 + p
            k = (1 + p) % 2
            wait_outs(k ^ 1)
            stage1(c + 1, k ^ 1)
            wait_gather(k)
            compute(c, k)
            fire_outs(c, k)
        return 0
    lax.fori_loop(0, (NCHUNK - 2) // 2, pair_body, 0)

    wait_outs(0)
    wait_gather(1)
    compute(NCHUNK - 1, 1)
    fire_outs(NCHUNK - 1, 1)
    wait_outs(1)


def kernel(x, table, W):
    x_flat = x.reshape(BL * FEAT)
    ws = (W.T / 255.0).reshape((FEAT - 1) * PROJ_DIM).astype(jnp.float32)
    out = _encode(x_flat, table, ws)
    return out.reshape(B, L, OUT_DIM)
